# baseline (speedup 1.0000x reference)
import functools

import numpy as np

import jax
import jax.numpy as jnp
from jax import lax
from jax.experimental import pallas as pl
from jax.experimental.pallas import tpu as pltpu, tpu_sc as plsc

N = 10000
D = 128
DH = D // 2
E = 320000

NC = 2
NS = 16
EDGES_T = 20480
E_PAD = NS * EDGES_T
CHUNK = 128
JJ = 2
ROWS_T = EDGES_T // CHUNK
STEPS = ROWS_T // JJ
N_PAD = 10240
ZR = 64
N_OUT_T = N_PAD // NS

_mesh = plsc.VectorSubcoreMesh(core_axis_name="c", subcore_axis_name="s")

_PI = np.zeros(D, np.int32)
for _base in range(0, D, 32):
    for _i in range(16):
        _PI[_base + _i] = _base + 2 * _i
        _PI[_base + 16 + _i] = _base + 2 * _i + 1
_INV = np.argsort(_PI)


def _agg_body(want_deg, *refs):
    if want_deg:
        (xn_hbm, src_hbm, dst_hbm, agg_out, deg_out,
         src_v, dst_v, raw_v, conv_v, ones_v, z16_v, agg_sh, deg_sh,
         gsem, ssem, dsem) = refs
    else:
        (xn_hbm, src_hbm, dst_hbm, agg_out,
         src_v, dst_v, raw_v, conv_v, agg_sh, gsem, ssem) = refs

    c = lax.axis_index("c")
    s = lax.axis_index("s")

    zz = jnp.zeros((16,), jnp.float32)

    def zrow(i, carry):
        for j in range(DH // 16):
            conv_v[i, pl.ds(j * 16, 16)] = zz
        return carry
    lax.fori_loop(0, ZR, zrow, 0)

    if want_deg:
        oo = jnp.ones((16,), jnp.float32)

        def orow(i, carry):
            ones_v[i, :] = oo
            return carry
        lax.fori_loop(0, CHUNK, orow, 0)

        def z16row(i, carry):
            z16_v[i, :] = zz
            return carry
        lax.fori_loop(0, ZR, z16row, 0)

    zbase = s * (N_PAD // NS)
    for k in range(N_PAD // NS // ZR):
        pltpu.sync_copy(conv_v.at[pl.ds(0, ZR)],
                        agg_sh.at[pl.ds(zbase + k * ZR, ZR)])
        if want_deg:
            pltpu.sync_copy(z16_v, deg_sh.at[pl.ds(zbase + k * ZR, ZR)])
    plsc.subcore_barrier()

    my_xn = xn_hbm.at[c]
    HALF = JJ * CHUNK
    SLAB = ROWS_T // 2
    SPH = SLAB // JJ

    def fire_gather(sidx, base):
        for j in range(JJ):
            pltpu.async_copy(
                my_xn.at[src_v.at[sidx * JJ + j]],
                raw_v.at[pl.ds(base + j * CHUNK, CHUNK)], gsem)

    def wait_gather(base):
        for j in range(JJ):
            pltpu.make_async_copy(
                my_xn.at[src_v.at[0]],
                raw_v.at[pl.ds(base + j * CHUNK, CHUNK)], gsem).wait()

    def fire_scatter(sidx, base):
        for j in range(JJ):
            pltpu.async_copy(
                conv_v.at[pl.ds(base + j * CHUNK, CHUNK)],
                agg_sh.at[dst_v.at[sidx * JJ + j]], ssem, add=True)

    def wait_scatter(base):
        for j in range(JJ):
            pltpu.make_async_copy(
                conv_v.at[pl.ds(base + j * CHUNK, CHUNK)],
                agg_sh.at[dst_v.at[0]], ssem).wait()

    def fire_deg(sidx):
        for j in range(JJ):
            pltpu.async_copy(ones_v, deg_sh.at[dst_v.at[sidx * JJ + j]],
                             dsem, add=True)

    def wait_deg():
        for j in range(JJ):
            pltpu.make_async_copy(ones_v, deg_sh.at[dst_v.at[0]],
                                  dsem).wait()

    def convert(base):
        def conv_row(i, carry):
            r = base + 4 * i
            for u in range(4):
                for g in range(DH // 32):
                    xx = raw_v[r + u, pl.ds(g * 32, 32)]
                    aa, bb = plsc.unpack(
                        xx, format=plsc.PackFormat.INTERLEAVED,
                        preferred_element_type=jnp.float32)
                    conv_v[r + u, pl.ds(g * 32, 16)] = aa
                    conv_v[r + u, pl.ds(g * 32 + 16, 16)] = bb
            return carry
        lax.fori_loop(0, HALF // 4, conv_row, 0)

    for phase in range(2):
        pbase = (s * ROWS_T + phase * SLAB)
        pltpu.sync_copy(src_hbm.at[pl.ds(pbase, SLAB)], src_v)
        pltpu.sync_copy(dst_hbm.at[pl.ds(pbase, SLAB)], dst_v)
        fire_gather(0, 0)

        def outer(t, carry):
            for b in range(2):
                ss = t * 2 + b
                cur = b * HALF
                oth = (1 - b) * HALF
                wait_gather(cur)

                @pl.when(ss + 1 < SPH)
                def _():
                    fire_gather(ss + 1, oth)

                @pl.when(ss >= 2)
                def _():
                    wait_scatter(cur)
                if want_deg:
                    @pl.when((ss >= 1) & (((ss - 1) & 1) == c))
                    def _():
                        wait_deg()
                convert(cur)
                fire_scatter(ss, cur)
                if want_deg:
                    @pl.when((ss & 1) == c)
                    def _():
                        fire_deg(ss)
            return carry
        lax.fori_loop(0, SPH // 2, outer, 0)
        wait_scatter(0)
        wait_scatter(HALF)
        if want_deg:
            @pl.when(((SPH - 1) & 1) == c)
            def _():
                wait_deg()
    plsc.subcore_barrier()

    obase = s * N_OUT_T
    pltpu.sync_copy(agg_sh.at[pl.ds(obase, N_OUT_T)],
                    agg_out.at[c, pl.ds(obase, N_OUT_T)])
    if want_deg:
        pltpu.sync_copy(deg_sh.at[pl.ds(obase, N_OUT_T)],
                        deg_out.at[c, pl.ds(obase, N_OUT_T)])


def _make_agg(want_deg):
    if want_deg:
        out_type = (jax.ShapeDtypeStruct((NC, N_PAD, DH), jnp.float32),
                    jax.ShapeDtypeStruct((NC, N_PAD, 16), jnp.float32))
        scratch = [
            pltpu.VMEM((ROWS_T // 2, CHUNK), jnp.int32),
            pltpu.VMEM((ROWS_T // 2, CHUNK), jnp.int32),
            pltpu.VMEM((2 * JJ * CHUNK, DH), jnp.bfloat16),
            pltpu.VMEM((2 * JJ * CHUNK, DH), jnp.float32),
            pltpu.VMEM((CHUNK, 16), jnp.float32),
            pltpu.VMEM((ZR, 16), jnp.float32),
            pltpu.VMEM_SHARED((N_PAD, DH), jnp.float32),
            pltpu.VMEM_SHARED((N_PAD, 16), jnp.float32),
            pltpu.SemaphoreType.DMA,
            pltpu.SemaphoreType.DMA,
            pltpu.SemaphoreType.DMA,
        ]
    else:
        out_type = jax.ShapeDtypeStruct((NC, N_PAD, DH), jnp.float32)
        scratch = [
            pltpu.VMEM((ROWS_T // 2, CHUNK), jnp.int32),
            pltpu.VMEM((ROWS_T // 2, CHUNK), jnp.int32),
            pltpu.VMEM((2 * JJ * CHUNK, DH), jnp.bfloat16),
            pltpu.VMEM((2 * JJ * CHUNK, DH), jnp.float32),
            pltpu.VMEM_SHARED((N_PAD, DH), jnp.float32),
            pltpu.SemaphoreType.DMA,
            pltpu.SemaphoreType.DMA,
        ]
    return pl.kernel(functools.partial(_agg_body, want_deg),
                     out_type=out_type, mesh=_mesh, scratch_types=scratch,
                     compiler_params=pltpu.CompilerParams(
                         use_tc_tiling_on_sc=False,
                         needs_layout_passes=False))


_agg_with_deg = _make_agg(True)
_agg_only = _make_agg(False)


_BLK = 2000
_GRID = N // _BLK


def _proj_body(x_ref, ws_ref, wn_ref, b_ref, xs_ref, xn_ref):
    x = x_ref[...]
    xs_ref[...] = jnp.dot(x, ws_ref[...],
                          preferred_element_type=jnp.float32) + b_ref[...]
    xn = jnp.dot(x, wn_ref[...], preferred_element_type=jnp.float32)
    xn_ref[0] = xn[:, :DH].astype(jnp.bfloat16)
    xn_ref[1] = xn[:, DH:].astype(jnp.bfloat16)


def _mid_body(xs_ref, aggp_ref, degp_ref, ws_ref, wn_ref, b_ref,
              xs2_ref, xn2_ref):
    agg = jnp.concatenate([aggp_ref[0], aggp_ref[1]], axis=1)
    deg = degp_ref[0, :, 0:1] + degp_ref[1, :, 0:1]
    h = jnp.maximum(xs_ref[...] + agg / jnp.maximum(deg, 1.0), 0.0)
    xs2_ref[...] = jnp.dot(h, ws_ref[...],
                           preferred_element_type=jnp.float32) + b_ref[...]
    xn2 = jnp.dot(h, wn_ref[...], preferred_element_type=jnp.float32)
    xn2_ref[0] = xn2[:, :DH].astype(jnp.bfloat16)
    xn2_ref[1] = xn2[:, DH:].astype(jnp.bfloat16)


def _final_body(xs_ref, aggp_ref, degp_ref, out_ref):
    agg = jnp.concatenate([aggp_ref[0], aggp_ref[1]], axis=1)
    deg = degp_ref[0, :, 0:1] + degp_ref[1, :, 0:1]
    out_ref[...] = jnp.maximum(xs_ref[...] + agg / jnp.maximum(deg, 1.0), 0.0)


_row_spec = pl.BlockSpec((_BLK, D), lambda i: (i, 0))
_w_spec = pl.BlockSpec((D, D), lambda i: (0, 0))
_b_spec = pl.BlockSpec((1, D), lambda i: (0, 0))
_xnh_spec = pl.BlockSpec((NC, _BLK, DH), lambda i: (0, i, 0))
_aggp_spec = pl.BlockSpec((NC, _BLK, DH), lambda i: (0, i, 0))
_degp_spec = pl.BlockSpec((NC, _BLK, 16), lambda i: (0, i, 0))
_nd = jax.ShapeDtypeStruct((N, D), jnp.float32)
_xnh_shape = jax.ShapeDtypeStruct((NC, N, DH), jnp.bfloat16)

_proj = pl.pallas_call(
    _proj_body, grid=(_GRID,),
    in_specs=[_row_spec, _w_spec, _w_spec, _b_spec],
    out_specs=[_row_spec, _xnh_spec], out_shape=[_nd, _xnh_shape])

_mid = pl.pallas_call(
    _mid_body, grid=(_GRID,),
    in_specs=[_row_spec, _aggp_spec, _degp_spec, _w_spec, _w_spec, _b_spec],
    out_specs=[_row_spec, _xnh_spec], out_shape=[_nd, _xnh_shape])

_final = pl.pallas_call(
    _final_body, grid=(_GRID,),
    in_specs=[_row_spec, _aggp_spec, _degp_spec],
    out_specs=_row_spec, out_shape=_nd)


def kernel(graph, features, W_self1, W_neigh1, b1, W_self2, W_neigh2, b2):
    src = graph[0].astype(jnp.int32)
    dst = graph[1].astype(jnp.int32)
    pad = E_PAD - src.shape[0]
    srcp = jnp.concatenate([src, jnp.zeros((pad,), jnp.int32)]
                           ).reshape(E_PAD // CHUNK, CHUNK)
    dstp = jnp.concatenate([dst, jnp.full((pad,), N, jnp.int32)]
                           ).reshape(E_PAD // CHUNK, CHUNK)
    b1r = b1.reshape(1, D)
    b2r = b2.reshape(1, D)

    pi = jnp.asarray(_PI)
    inv = jnp.asarray(_INV)
    xs1, xn1 = _proj(features, W_self1[:, pi], W_neigh1, b1r[:, pi])
    aggp1, degp = _agg_with_deg(xn1, srcp, dstp)
    xs2, xn2 = _mid(xs1, aggp1, degp, W_self2[pi][:, pi], W_neigh2[pi],
                    b2r[:, pi])
    aggp2 = _agg_only(xn2, srcp, dstp)
    return _final(xs2, aggp2, degp)[:, inv]


# speedup vs baseline: 1.0071x; 1.0071x over previous
import functools

import numpy as np

import jax
import jax.numpy as jnp
from jax import lax
from jax.experimental import pallas as pl
from jax.experimental.pallas import tpu as pltpu, tpu_sc as plsc

N = 10000
D = 128
DH = D // 2
E = 320000

NC = 2
NS = 16
EDGES_T = 20480
E_PAD = NS * EDGES_T
CHUNK = 128
JJ = 2
ROWS_T = EDGES_T // CHUNK
STEPS = ROWS_T // JJ
N_PAD = 10240
ZR = 64
N_OUT_T = N_PAD // NS

_mesh = plsc.VectorSubcoreMesh(core_axis_name="c", subcore_axis_name="s")

_PI = np.zeros(D, np.int32)
for _base in range(0, D, 32):
    for _i in range(16):
        _PI[_base + _i] = _base + 2 * _i
        _PI[_base + 16 + _i] = _base + 2 * _i + 1
_INV = np.argsort(_PI)


def _agg_body(want_deg, *refs):
    if want_deg:
        (xn_hbm, src_hbm, dst_hbm, agg_out, deg_out,
         src_v, dst_v, raw_v, conv_v, ones_v, z16_v, agg_sh, deg_sh,
         gsem, ssem0, ssem1, dsem) = refs
    else:
        (xn_hbm, src_hbm, dst_hbm, agg_out,
         src_v, dst_v, raw_v, conv_v, agg_sh, gsem, ssem0, ssem1) = refs

    c = lax.axis_index("c")
    s = lax.axis_index("s")

    zz = jnp.zeros((16,), jnp.float32)

    def zrow(i, carry):
        for j in range(DH // 16):
            conv_v[i, pl.ds(j * 16, 16)] = zz
        return carry
    lax.fori_loop(0, ZR, zrow, 0)

    if want_deg:
        oo = jnp.ones((16,), jnp.float32)

        def orow(i, carry):
            ones_v[i, :] = oo
            return carry
        lax.fori_loop(0, CHUNK, orow, 0)

        def z16row(i, carry):
            z16_v[i, :] = zz
            return carry
        lax.fori_loop(0, ZR, z16row, 0)

    zbase = s * (N_PAD // NS)
    for k in range(N_PAD // NS // ZR):
        pltpu.sync_copy(conv_v.at[pl.ds(0, ZR)],
                        agg_sh.at[pl.ds(zbase + k * ZR, ZR)])
        if want_deg:
            pltpu.sync_copy(z16_v, deg_sh.at[pl.ds(zbase + k * ZR, ZR)])
    plsc.subcore_barrier()

    my_xn = xn_hbm.at[c]
    HALF = JJ * CHUNK
    SLAB = ROWS_T // 2
    SPH = SLAB // JJ

    def fire_gather(sidx, base):
        for j in range(JJ):
            pltpu.async_copy(
                my_xn.at[src_v.at[sidx * JJ + j]],
                raw_v.at[pl.ds(base + j * CHUNK, CHUNK)], gsem)

    def wait_gather(base):
        for j in range(JJ):
            pltpu.make_async_copy(
                my_xn.at[src_v.at[0]],
                raw_v.at[pl.ds(base + j * CHUNK, CHUNK)], gsem).wait()

    def fire_scatter(sidx, base, sem):
        for j in range(JJ):
            pltpu.async_copy(
                conv_v.at[pl.ds(base + j * CHUNK, CHUNK)],
                agg_sh.at[dst_v.at[sidx * JJ + j]], sem, add=True)

    def wait_scatter(base, sem):
        for j in range(JJ):
            pltpu.make_async_copy(
                conv_v.at[pl.ds(base + j * CHUNK, CHUNK)],
                agg_sh.at[dst_v.at[0]], sem).wait()

    def fire_deg(sidx):
        for j in range(JJ):
            pltpu.async_copy(ones_v, deg_sh.at[dst_v.at[sidx * JJ + j]],
                             dsem, add=True)

    def wait_deg():
        for j in range(JJ):
            pltpu.make_async_copy(ones_v, deg_sh.at[dst_v.at[0]],
                                  dsem).wait()

    def convert(base):
        def conv_row(i, carry):
            r = base + 4 * i
            for u in range(4):
                for g in range(DH // 32):
                    xx = raw_v[r + u, pl.ds(g * 32, 32)]
                    aa, bb = plsc.unpack(
                        xx, format=plsc.PackFormat.INTERLEAVED,
                        preferred_element_type=jnp.float32)
                    conv_v[r + u, pl.ds(g * 32, 16)] = aa
                    conv_v[r + u, pl.ds(g * 32 + 16, 16)] = bb
            return carry
        lax.fori_loop(0, HALF // 4, conv_row, 0)

    for phase in range(2):
        pbase = (s * ROWS_T + phase * SLAB)
        pltpu.sync_copy(src_hbm.at[pl.ds(pbase, SLAB)], src_v)
        pltpu.sync_copy(dst_hbm.at[pl.ds(pbase, SLAB)], dst_v)
        fire_gather(0, 0)

        def outer(t, carry):
            for b in range(2):
                ss = t * 2 + b
                cur = b * HALF
                oth = (1 - b) * HALF
                wait_gather(cur)

                @pl.when(ss + 1 < SPH)
                def _():
                    fire_gather(ss + 1, oth)

                ssem = ssem0 if b == 0 else ssem1

                @pl.when(ss >= 2)
                def _():
                    wait_scatter(cur, ssem)
                if want_deg:
                    @pl.when((ss >= 1) & (((ss - 1) & 1) == c))
                    def _():
                        wait_deg()
                convert(cur)
                fire_scatter(ss, cur, ssem)
                if want_deg:
                    @pl.when((ss & 1) == c)
                    def _():
                        fire_deg(ss)
            return carry
        lax.fori_loop(0, SPH // 2, outer, 0)
        wait_scatter(0, ssem0)
        wait_scatter(HALF, ssem1)
        if want_deg:
            @pl.when(((SPH - 1) & 1) == c)
            def _():
                wait_deg()
    plsc.subcore_barrier()

    obase = s * N_OUT_T
    pltpu.sync_copy(agg_sh.at[pl.ds(obase, N_OUT_T)],
                    agg_out.at[c, pl.ds(obase, N_OUT_T)])
    if want_deg:
        pltpu.sync_copy(deg_sh.at[pl.ds(obase, N_OUT_T)],
                        deg_out.at[c, pl.ds(obase, N_OUT_T)])


def _make_agg(want_deg):
    if want_deg:
        out_type = (jax.ShapeDtypeStruct((NC, N_PAD, DH), jnp.float32),
                    jax.ShapeDtypeStruct((NC, N_PAD, 16), jnp.float32))
        scratch = [
            pltpu.VMEM((ROWS_T // 2, CHUNK), jnp.int32),
            pltpu.VMEM((ROWS_T // 2, CHUNK), jnp.int32),
            pltpu.VMEM((2 * JJ * CHUNK, DH), jnp.bfloat16),
            pltpu.VMEM((2 * JJ * CHUNK, DH), jnp.float32),
            pltpu.VMEM((CHUNK, 16), jnp.float32),
            pltpu.VMEM((ZR, 16), jnp.float32),
            pltpu.VMEM_SHARED((N_PAD, DH), jnp.float32),
            pltpu.VMEM_SHARED((N_PAD, 16), jnp.float32),
            pltpu.SemaphoreType.DMA,
            pltpu.SemaphoreType.DMA,
            pltpu.SemaphoreType.DMA,
            pltpu.SemaphoreType.DMA,
        ]
    else:
        out_type = jax.ShapeDtypeStruct((NC, N_PAD, DH), jnp.float32)
        scratch = [
            pltpu.VMEM((ROWS_T // 2, CHUNK), jnp.int32),
            pltpu.VMEM((ROWS_T // 2, CHUNK), jnp.int32),
            pltpu.VMEM((2 * JJ * CHUNK, DH), jnp.bfloat16),
            pltpu.VMEM((2 * JJ * CHUNK, DH), jnp.float32),
            pltpu.VMEM_SHARED((N_PAD, DH), jnp.float32),
            pltpu.SemaphoreType.DMA,
            pltpu.SemaphoreType.DMA,
            pltpu.SemaphoreType.DMA,
        ]
    return pl.kernel(functools.partial(_agg_body, want_deg),
                     out_type=out_type, mesh=_mesh, scratch_types=scratch,
                     compiler_params=pltpu.CompilerParams(
                         use_tc_tiling_on_sc=False,
                         needs_layout_passes=False))


_agg_with_deg = _make_agg(True)
_agg_only = _make_agg(False)


_BLK = 2000
_GRID = N // _BLK


def _proj_body(x_ref, ws_ref, wn_ref, b_ref, xs_ref, xn_ref):
    x = x_ref[...]
    xs_ref[...] = jnp.dot(x, ws_ref[...],
                          preferred_element_type=jnp.float32) + b_ref[...]
    xn = jnp.dot(x, wn_ref[...], preferred_element_type=jnp.float32)
    xn_ref[0] = xn[:, :DH].astype(jnp.bfloat16)
    xn_ref[1] = xn[:, DH:].astype(jnp.bfloat16)


def _mid_body(xs_ref, aggp_ref, degp_ref, ws_ref, wn_ref, b_ref,
              xs2_ref, xn2_ref):
    agg = jnp.concatenate([aggp_ref[0], aggp_ref[1]], axis=1)
    deg = degp_ref[0, :, 0:1] + degp_ref[1, :, 0:1]
    h = jnp.maximum(xs_ref[...] + agg / jnp.maximum(deg, 1.0), 0.0)
    xs2_ref[...] = jnp.dot(h, ws_ref[...],
                           preferred_element_type=jnp.float32) + b_ref[...]
    xn2 = jnp.dot(h, wn_ref[...], preferred_element_type=jnp.float32)
    xn2_ref[0] = xn2[:, :DH].astype(jnp.bfloat16)
    xn2_ref[1] = xn2[:, DH:].astype(jnp.bfloat16)


def _final_body(xs_ref, aggp_ref, degp_ref, out_ref):
    agg = jnp.concatenate([aggp_ref[0], aggp_ref[1]], axis=1)
    deg = degp_ref[0, :, 0:1] + degp_ref[1, :, 0:1]
    out_ref[...] = jnp.maximum(xs_ref[...] + agg / jnp.maximum(deg, 1.0), 0.0)


_row_spec = pl.BlockSpec((_BLK, D), lambda i: (i, 0))
_w_spec = pl.BlockSpec((D, D), lambda i: (0, 0))
_b_spec = pl.BlockSpec((1, D), lambda i: (0, 0))
_xnh_spec = pl.BlockSpec((NC, _BLK, DH), lambda i: (0, i, 0))
_aggp_spec = pl.BlockSpec((NC, _BLK, DH), lambda i: (0, i, 0))
_degp_spec = pl.BlockSpec((NC, _BLK, 16), lambda i: (0, i, 0))
_nd = jax.ShapeDtypeStruct((N, D), jnp.float32)
_xnh_shape = jax.ShapeDtypeStruct((NC, N, DH), jnp.bfloat16)

_proj = pl.pallas_call(
    _proj_body, grid=(_GRID,),
    in_specs=[_row_spec, _w_spec, _w_spec, _b_spec],
    out_specs=[_row_spec, _xnh_spec], out_shape=[_nd, _xnh_shape])

_mid = pl.pallas_call(
    _mid_body, grid=(_GRID,),
    in_specs=[_row_spec, _aggp_spec, _degp_spec, _w_spec, _w_spec, _b_spec],
    out_specs=[_row_spec, _xnh_spec], out_shape=[_nd, _xnh_shape])

_final = pl.pallas_call(
    _final_body, grid=(_GRID,),
    in_specs=[_row_spec, _aggp_spec, _degp_spec],
    out_specs=_row_spec, out_shape=_nd)


def kernel(graph, features, W_self1, W_neigh1, b1, W_self2, W_neigh2, b2):
    src = graph[0].astype(jnp.int32)
    dst = graph[1].astype(jnp.int32)
    pad = E_PAD - src.shape[0]
    srcp = jnp.concatenate([src, jnp.zeros((pad,), jnp.int32)]
                           ).reshape(E_PAD // CHUNK, CHUNK)
    dstp = jnp.concatenate([dst, jnp.full((pad,), N, jnp.int32)]
                           ).reshape(E_PAD // CHUNK, CHUNK)
    b1r = b1.reshape(1, D)
    b2r = b2.reshape(1, D)

    pi = jnp.asarray(_PI)
    inv = jnp.asarray(_INV)
    xs1, xn1 = _proj(features, W_self1[:, pi], W_neigh1, b1r[:, pi])
    aggp1, degp = _agg_with_deg(xn1, srcp, dstp)
    xs2, xn2 = _mid(xs1, aggp1, degp, W_self2[pi][:, pi], W_neigh2[pi],
                    b2r[:, pi])
    aggp2 = _agg_only(xn2, srcp, dstp)
    return _final(xs2, aggp2, degp)[:, inv]


# speedup vs baseline: 1.7031x; 1.6911x over previous
import functools

import jax
import jax.numpy as jnp
from jax import lax
from jax.experimental import pallas as pl
from jax.experimental.pallas import tpu as pltpu, tpu_sc as plsc

N = 10000
D = 128
DH = D // 2
E = 320000

NC = 2
NS = 16
EDGES_T = 20480
E_PAD = NS * EDGES_T
CHUNK = 128
JJ = 4
ROWS_T = EDGES_T // CHUNK
STEPS = ROWS_T // JJ
N_PAD = 10240
ZR = 64
N_OUT_T = N_PAD // NS

_mesh = plsc.VectorSubcoreMesh(core_axis_name="c", subcore_axis_name="s")


def _agg_body(want_deg, *refs):
    if want_deg:
        (xn_hbm, src_hbm, dst_hbm, agg_out, deg_out,
         src_v, dst_v, raw_v, ones_v, z16_v, agg_sh, deg_sh,
         gsem, ssem0, ssem1, dsem) = refs
    else:
        (xn_hbm, src_hbm, dst_hbm, agg_out,
         src_v, dst_v, raw_v, agg_sh, gsem, ssem0, ssem1) = refs

    c = lax.axis_index("c")
    s = lax.axis_index("s")

    zz32 = jnp.zeros((32,), jnp.bfloat16)

    def zrow(i, carry):
        for j in range(DH // 32):
            raw_v[i, pl.ds(j * 32, 32)] = zz32
        return carry
    lax.fori_loop(0, ZR, zrow, 0)

    if want_deg:
        zz = jnp.zeros((16,), jnp.float32)
        oo = jnp.ones((16,), jnp.float32)

        def orow(i, carry):
            ones_v[i, :] = oo
            return carry
        lax.fori_loop(0, CHUNK, orow, 0)

        def z16row(i, carry):
            z16_v[i, :] = zz
            return carry
        lax.fori_loop(0, ZR, z16row, 0)

    zbase = s * (N_PAD // NS)
    for k in range(N_PAD // NS // ZR):
        pltpu.sync_copy(raw_v.at[pl.ds(0, ZR)],
                        agg_sh.at[pl.ds(zbase + k * ZR, ZR)])
        if want_deg:
            pltpu.sync_copy(z16_v, deg_sh.at[pl.ds(zbase + k * ZR, ZR)])
    plsc.subcore_barrier()

    pltpu.sync_copy(src_hbm.at[pl.ds(s * ROWS_T, ROWS_T)], src_v)
    pltpu.sync_copy(dst_hbm.at[pl.ds(s * ROWS_T, ROWS_T)], dst_v)

    my_xn = xn_hbm.at[c]
    HALF = JJ * CHUNK

    def fire_gather(sidx, base):
        for j in range(JJ):
            pltpu.async_copy(
                my_xn.at[src_v.at[sidx * JJ + j]],
                raw_v.at[pl.ds(base + j * CHUNK, CHUNK)], gsem)

    def wait_gather(base):
        for j in range(JJ):
            pltpu.make_async_copy(
                my_xn.at[src_v.at[0]],
                raw_v.at[pl.ds(base + j * CHUNK, CHUNK)], gsem).wait()

    def fire_scatter(sidx, base, sem):
        for j in range(JJ):
            pltpu.async_copy(
                raw_v.at[pl.ds(base + j * CHUNK, CHUNK)],
                agg_sh.at[dst_v.at[sidx * JJ + j]], sem, add=True)

    def wait_scatter(base, sem):
        for j in range(JJ):
            pltpu.make_async_copy(
                raw_v.at[pl.ds(base + j * CHUNK, CHUNK)],
                agg_sh.at[dst_v.at[0]], sem).wait()

    def fire_deg(sidx):
        for j in range(JJ):
            pltpu.async_copy(ones_v, deg_sh.at[dst_v.at[sidx * JJ + j]],
                             dsem, add=True)

    def wait_deg():
        for j in range(JJ):
            pltpu.make_async_copy(ones_v, deg_sh.at[dst_v.at[0]],
                                  dsem).wait()

    fire_gather(0, 0)

    def outer(t, carry):
        for b in range(2):
            s2 = t * 2 + b
            cur = b * HALF
            oth = (1 - b) * HALF
            semc = ssem0 if b == 0 else ssem1
            semo = ssem1 if b == 0 else ssem0
            wait_gather(cur)

            @pl.when(s2 >= 1)
            def _():
                wait_scatter(oth, semo)
            if want_deg:
                @pl.when((s2 >= 1) & (((s2 - 1) & 1) == c))
                def _():
                    wait_deg()

            @pl.when(s2 + 1 < STEPS)
            def _():
                fire_gather(s2 + 1, oth)
            fire_scatter(s2, cur, semc)
            if want_deg:
                @pl.when((s2 & 1) == c)
                def _():
                    fire_deg(s2)
        return carry
    lax.fori_loop(0, STEPS // 2, outer, 0)
    wait_scatter(HALF, ssem1)
    if want_deg:
        @pl.when(((STEPS - 1) & 1) == c)
        def _():
            wait_deg()

    plsc.subcore_barrier()

    obase = s * N_OUT_T
    pltpu.sync_copy(agg_sh.at[pl.ds(obase, N_OUT_T)],
                    agg_out.at[c, pl.ds(obase, N_OUT_T)])
    if want_deg:
        pltpu.sync_copy(deg_sh.at[pl.ds(obase, N_OUT_T)],
                        deg_out.at[c, pl.ds(obase, N_OUT_T)])


def _make_agg(want_deg):
    if want_deg:
        out_type = (jax.ShapeDtypeStruct((NC, N_PAD, DH), jnp.bfloat16),
                    jax.ShapeDtypeStruct((NC, N_PAD, 16), jnp.float32))
        scratch = [
            pltpu.VMEM((ROWS_T, CHUNK), jnp.int32),
            pltpu.VMEM((ROWS_T, CHUNK), jnp.int32),
            pltpu.VMEM((2 * JJ * CHUNK, DH), jnp.bfloat16),
            pltpu.VMEM((CHUNK, 16), jnp.float32),
            pltpu.VMEM((ZR, 16), jnp.float32),
            pltpu.VMEM_SHARED((N_PAD, DH), jnp.bfloat16),
            pltpu.VMEM_SHARED((N_PAD, 16), jnp.float32),
            pltpu.SemaphoreType.DMA,
            pltpu.SemaphoreType.DMA,
            pltpu.SemaphoreType.DMA,
            pltpu.SemaphoreType.DMA,
        ]
    else:
        out_type = jax.ShapeDtypeStruct((NC, N_PAD, DH), jnp.bfloat16)
        scratch = [
            pltpu.VMEM((ROWS_T, CHUNK), jnp.int32),
            pltpu.VMEM((ROWS_T, CHUNK), jnp.int32),
            pltpu.VMEM((2 * JJ * CHUNK, DH), jnp.bfloat16),
            pltpu.VMEM_SHARED((N_PAD, DH), jnp.bfloat16),
            pltpu.SemaphoreType.DMA,
            pltpu.SemaphoreType.DMA,
            pltpu.SemaphoreType.DMA,
        ]
    return pl.kernel(functools.partial(_agg_body, want_deg),
                     out_type=out_type, mesh=_mesh, scratch_types=scratch,
                     compiler_params=pltpu.CompilerParams(
                         use_tc_tiling_on_sc=False,
                         needs_layout_passes=False))


_agg_with_deg = _make_agg(True)
_agg_only = _make_agg(False)


_BLK = 2000
_GRID = N // _BLK


def _proj_body(x_ref, ws_ref, wn_ref, b_ref, xs_ref, xn_ref):
    x = x_ref[...]
    xs_ref[...] = jnp.dot(x, ws_ref[...],
                          preferred_element_type=jnp.float32) + b_ref[...]
    xn = jnp.dot(x, wn_ref[...], preferred_element_type=jnp.float32)
    xn_ref[0] = xn[:, :DH].astype(jnp.bfloat16)
    xn_ref[1] = xn[:, DH:].astype(jnp.bfloat16)


def _mid_body(xs_ref, aggp_ref, degp_ref, ws_ref, wn_ref, b_ref,
              xs2_ref, xn2_ref):
    agg = jnp.concatenate([aggp_ref[0].astype(jnp.float32),
                           aggp_ref[1].astype(jnp.float32)], axis=1)
    deg = degp_ref[0, :, 0:1] + degp_ref[1, :, 0:1]
    h = jnp.maximum(xs_ref[...] + agg / jnp.maximum(deg, 1.0), 0.0)
    xs2_ref[...] = jnp.dot(h, ws_ref[...],
                           preferred_element_type=jnp.float32) + b_ref[...]
    xn2 = jnp.dot(h, wn_ref[...], preferred_element_type=jnp.float32)
    xn2_ref[0] = xn2[:, :DH].astype(jnp.bfloat16)
    xn2_ref[1] = xn2[:, DH:].astype(jnp.bfloat16)


def _final_body(xs_ref, aggp_ref, degp_ref, out_ref):
    agg = jnp.concatenate([aggp_ref[0].astype(jnp.float32),
                           aggp_ref[1].astype(jnp.float32)], axis=1)
    deg = degp_ref[0, :, 0:1] + degp_ref[1, :, 0:1]
    out_ref[...] = jnp.maximum(xs_ref[...] + agg / jnp.maximum(deg, 1.0), 0.0)


_row_spec = pl.BlockSpec((_BLK, D), lambda i: (i, 0))
_w_spec = pl.BlockSpec((D, D), lambda i: (0, 0))
_b_spec = pl.BlockSpec((1, D), lambda i: (0, 0))
_xnh_spec = pl.BlockSpec((NC, _BLK, DH), lambda i: (0, i, 0))
_aggp_spec = pl.BlockSpec((NC, _BLK, DH), lambda i: (0, i, 0))
_degp_spec = pl.BlockSpec((NC, _BLK, 16), lambda i: (0, i, 0))
_nd = jax.ShapeDtypeStruct((N, D), jnp.float32)
_xnh_shape = jax.ShapeDtypeStruct((NC, N, DH), jnp.bfloat16)

_proj = pl.pallas_call(
    _proj_body, grid=(_GRID,),
    in_specs=[_row_spec, _w_spec, _w_spec, _b_spec],
    out_specs=[_row_spec, _xnh_spec], out_shape=[_nd, _xnh_shape])

_mid = pl.pallas_call(
    _mid_body, grid=(_GRID,),
    in_specs=[_row_spec, _aggp_spec, _degp_spec, _w_spec, _w_spec, _b_spec],
    out_specs=[_row_spec, _xnh_spec], out_shape=[_nd, _xnh_shape])

_final = pl.pallas_call(
    _final_body, grid=(_GRID,),
    in_specs=[_row_spec, _aggp_spec, _degp_spec],
    out_specs=_row_spec, out_shape=_nd)


def kernel(graph, features, W_self1, W_neigh1, b1, W_self2, W_neigh2, b2):
    src = graph[0].astype(jnp.int32)
    dst = graph[1].astype(jnp.int32)
    pad = E_PAD - src.shape[0]
    srcp = jnp.concatenate([src, jnp.zeros((pad,), jnp.int32)]
                           ).reshape(E_PAD // CHUNK, CHUNK)
    dstp = jnp.concatenate([dst, jnp.full((pad,), N, jnp.int32)]
                           ).reshape(E_PAD // CHUNK, CHUNK)
    b1r = b1.reshape(1, D)
    b2r = b2.reshape(1, D)

    xs1, xn1 = _proj(features, W_self1, W_neigh1, b1r)
    aggp1, degp = _agg_with_deg(xn1, srcp, dstp)
    xs2, xn2 = _mid(xs1, aggp1, degp, W_self2, W_neigh2, b2r)
    aggp2 = _agg_only(xn2, srcp, dstp)
    return _final(xs2, aggp2, degp)


# speedup vs baseline: 1.7237x; 1.0121x over previous
import functools

import jax
import jax.numpy as jnp
from jax import lax
from jax.experimental import pallas as pl
from jax.experimental.pallas import tpu as pltpu, tpu_sc as plsc

N = 10000
D = 128
DH = D // 2
E = 320000

NC = 2
NS = 16
EDGES_T = 20480
E_PAD = NS * EDGES_T
CHUNK = 128
JJ = 4
ROWS_T = EDGES_T // CHUNK
STEPS = ROWS_T // JJ
N_PAD = 10240
ZR = 64
N_OUT_T = N_PAD // NS

_mesh = plsc.VectorSubcoreMesh(core_axis_name="c", subcore_axis_name="s")


def _agg_body(want_deg, *refs):
    if want_deg:
        (xn_hbm, src_hbm, dst_hbm, agg_out, deg_out,
         src_v, dst_v, raw_v, ones_v, z16_v, agg_sh, deg_sh,
         gsem, ssem0, ssem1, dsem) = refs
    else:
        (xn_hbm, src_hbm, dst_hbm, agg_out,
         src_v, dst_v, raw_v, agg_sh, gsem, ssem0, ssem1) = refs

    c = lax.axis_index("c")
    s = lax.axis_index("s")

    zz32 = jnp.zeros((32,), jnp.bfloat16)

    def zrow(i, carry):
        for j in range(DH // 32):
            raw_v[i, pl.ds(j * 32, 32)] = zz32
        return carry
    lax.fori_loop(0, ZR, zrow, 0)

    if want_deg:
        zz = jnp.zeros((16,), jnp.float32)
        oo = jnp.ones((16,), jnp.float32)

        def orow(i, carry):
            ones_v[i, :] = oo
            return carry
        lax.fori_loop(0, CHUNK, orow, 0)

        def z16row(i, carry):
            z16_v[i, :] = zz
            return carry
        lax.fori_loop(0, ZR, z16row, 0)

    zbase = s * (N_PAD // NS)
    for k in range(N_PAD // NS // ZR):
        pltpu.sync_copy(raw_v.at[pl.ds(0, ZR)],
                        agg_sh.at[pl.ds(zbase + k * ZR, ZR)])
        if want_deg:
            pltpu.sync_copy(z16_v, deg_sh.at[pl.ds(zbase + k * ZR, ZR)])
    plsc.subcore_barrier()

    pltpu.sync_copy(src_hbm.at[pl.ds(s * ROWS_T, ROWS_T)], src_v)
    pltpu.sync_copy(dst_hbm.at[pl.ds(s * ROWS_T, ROWS_T)], dst_v)

    my_xn = xn_hbm.at[c]
    HALF = JJ * CHUNK

    def fire_gather(sidx, base):
        for j in range(JJ):
            pltpu.async_copy(
                my_xn.at[src_v.at[sidx * JJ + j]],
                raw_v.at[pl.ds(base + j * CHUNK, CHUNK)], gsem)

    def wait_gather(base):
        for j in range(JJ):
            pltpu.make_async_copy(
                my_xn.at[src_v.at[0]],
                raw_v.at[pl.ds(base + j * CHUNK, CHUNK)], gsem).wait()

    def fire_scatter(sidx, base, sem):
        for j in range(JJ):
            pltpu.async_copy(
                raw_v.at[pl.ds(base + j * CHUNK, CHUNK)],
                agg_sh.at[dst_v.at[sidx * JJ + j]], sem, add=True)

    def wait_scatter(base, sem):
        for j in range(JJ):
            pltpu.make_async_copy(
                raw_v.at[pl.ds(base + j * CHUNK, CHUNK)],
                agg_sh.at[dst_v.at[0]], sem).wait()

    def fire_deg(sidx):
        for j in range(JJ):
            pltpu.async_copy(ones_v, deg_sh.at[dst_v.at[sidx * JJ + j]],
                             dsem, add=True)

    def wait_deg():
        for j in range(JJ):
            pltpu.make_async_copy(ones_v, deg_sh.at[dst_v.at[0]],
                                  dsem).wait()

    fire_gather(0, 0)

    def outer(t, carry):
        for b in range(2):
            s2 = t * 2 + b
            cur = b * HALF
            oth = (1 - b) * HALF
            semc = ssem0 if b == 0 else ssem1
            semo = ssem1 if b == 0 else ssem0
            wait_gather(cur)

            @pl.when(s2 >= 1)
            def _():
                wait_scatter(oth, semo)
            if want_deg:
                @pl.when((s2 >= 1) & (((s2 - 1) & 1) == c))
                def _():
                    wait_deg()

            @pl.when(s2 + 1 < STEPS)
            def _():
                fire_gather(s2 + 1, oth)
            fire_scatter(s2, cur, semc)
            if want_deg:
                @pl.when((s2 & 1) == c)
                def _():
                    fire_deg(s2)
        return carry
    lax.fori_loop(0, STEPS // 2, outer, 0)
    wait_scatter(HALF, ssem1)
    if want_deg:
        @pl.when(((STEPS - 1) & 1) == c)
        def _():
            wait_deg()

    plsc.subcore_barrier()

    obase = s * N_OUT_T
    pltpu.sync_copy(agg_sh.at[pl.ds(obase, N_OUT_T)],
                    agg_out.at[c, pl.ds(obase, N_OUT_T)])
    if want_deg:
        pltpu.sync_copy(deg_sh.at[pl.ds(obase, N_OUT_T)],
                        deg_out.at[c, pl.ds(obase, N_OUT_T)])


def _make_agg(want_deg):
    if want_deg:
        out_type = (jax.ShapeDtypeStruct((NC, N_PAD, DH), jnp.bfloat16),
                    jax.ShapeDtypeStruct((NC, N_PAD, 16), jnp.float32))
        scratch = [
            pltpu.VMEM((ROWS_T, CHUNK), jnp.int32),
            pltpu.VMEM((ROWS_T, CHUNK), jnp.int32),
            pltpu.VMEM((2 * JJ * CHUNK, DH), jnp.bfloat16),
            pltpu.VMEM((CHUNK, 16), jnp.float32),
            pltpu.VMEM((ZR, 16), jnp.float32),
            pltpu.VMEM_SHARED((N_PAD, DH), jnp.bfloat16),
            pltpu.VMEM_SHARED((N_PAD, 16), jnp.float32),
            pltpu.SemaphoreType.DMA,
            pltpu.SemaphoreType.DMA,
            pltpu.SemaphoreType.DMA,
            pltpu.SemaphoreType.DMA,
        ]
    else:
        out_type = jax.ShapeDtypeStruct((NC, N_PAD, DH), jnp.bfloat16)
        scratch = [
            pltpu.VMEM((ROWS_T, CHUNK), jnp.int32),
            pltpu.VMEM((ROWS_T, CHUNK), jnp.int32),
            pltpu.VMEM((2 * JJ * CHUNK, DH), jnp.bfloat16),
            pltpu.VMEM_SHARED((N_PAD, DH), jnp.bfloat16),
            pltpu.SemaphoreType.DMA,
            pltpu.SemaphoreType.DMA,
            pltpu.SemaphoreType.DMA,
        ]
    return pl.kernel(functools.partial(_agg_body, want_deg),
                     out_type=out_type, mesh=_mesh, scratch_types=scratch,
                     compiler_params=pltpu.CompilerParams(
                         use_tc_tiling_on_sc=False,
                         needs_layout_passes=False))


_agg_with_deg = _make_agg(True)
_agg_only = _make_agg(False)


_BLK = 5000
_GRID = N // _BLK


def _proj_body(x_ref, ws_ref, wn_ref, b_ref, xs_ref, xn_ref):
    x = x_ref[...]
    xs_ref[...] = jnp.dot(x, ws_ref[...],
                          preferred_element_type=jnp.float32) + b_ref[...]
    xn = jnp.dot(x, wn_ref[...], preferred_element_type=jnp.float32)
    xn_ref[0] = xn[:, :DH].astype(jnp.bfloat16)
    xn_ref[1] = xn[:, DH:].astype(jnp.bfloat16)


def _mid_body(xs_ref, aggp_ref, degp_ref, ws_ref, wn_ref, b_ref,
              xs2_ref, xn2_ref):
    agg = jnp.concatenate([aggp_ref[0].astype(jnp.float32),
                           aggp_ref[1].astype(jnp.float32)], axis=1)
    deg = degp_ref[0, :, 0:1] + degp_ref[1, :, 0:1]
    h = jnp.maximum(xs_ref[...] + agg / jnp.maximum(deg, 1.0), 0.0)
    xs2_ref[...] = jnp.dot(h, ws_ref[...],
                           preferred_element_type=jnp.float32) + b_ref[...]
    xn2 = jnp.dot(h, wn_ref[...], preferred_element_type=jnp.float32)
    xn2_ref[0] = xn2[:, :DH].astype(jnp.bfloat16)
    xn2_ref[1] = xn2[:, DH:].astype(jnp.bfloat16)


def _final_body(xs_ref, aggp_ref, degp_ref, out_ref):
    agg = jnp.concatenate([aggp_ref[0].astype(jnp.float32),
                           aggp_ref[1].astype(jnp.float32)], axis=1)
    deg = degp_ref[0, :, 0:1] + degp_ref[1, :, 0:1]
    out_ref[...] = jnp.maximum(xs_ref[...] + agg / jnp.maximum(deg, 1.0), 0.0)


_row_spec = pl.BlockSpec((_BLK, D), lambda i: (i, 0))
_w_spec = pl.BlockSpec((D, D), lambda i: (0, 0))
_b_spec = pl.BlockSpec((1, D), lambda i: (0, 0))
_xnh_spec = pl.BlockSpec((NC, _BLK, DH), lambda i: (0, i, 0))
_aggp_spec = pl.BlockSpec((NC, _BLK, DH), lambda i: (0, i, 0))
_degp_spec = pl.BlockSpec((NC, _BLK, 16), lambda i: (0, i, 0))
_nd = jax.ShapeDtypeStruct((N, D), jnp.float32)
_xnh_shape = jax.ShapeDtypeStruct((NC, N, DH), jnp.bfloat16)

_proj = pl.pallas_call(
    _proj_body, grid=(_GRID,),
    in_specs=[_row_spec, _w_spec, _w_spec, _b_spec],
    out_specs=[_row_spec, _xnh_spec], out_shape=[_nd, _xnh_shape])

_mid = pl.pallas_call(
    _mid_body, grid=(_GRID,),
    in_specs=[_row_spec, _aggp_spec, _degp_spec, _w_spec, _w_spec, _b_spec],
    out_specs=[_row_spec, _xnh_spec], out_shape=[_nd, _xnh_shape])

_final = pl.pallas_call(
    _final_body, grid=(_GRID,),
    in_specs=[_row_spec, _aggp_spec, _degp_spec],
    out_specs=_row_spec, out_shape=_nd)


def kernel(graph, features, W_self1, W_neigh1, b1, W_self2, W_neigh2, b2):
    src = graph[0].astype(jnp.int32)
    dst = graph[1].astype(jnp.int32)
    pad = E_PAD - src.shape[0]
    srcp = jnp.concatenate([src, jnp.zeros((pad,), jnp.int32)]
                           ).reshape(E_PAD // CHUNK, CHUNK)
    dstp = jnp.concatenate([dst, jnp.full((pad,), N, jnp.int32)]
                           ).reshape(E_PAD // CHUNK, CHUNK)
    b1r = b1.reshape(1, D)
    b2r = b2.reshape(1, D)

    xs1, xn1 = _proj(features, W_self1, W_neigh1, b1r)
    aggp1, degp = _agg_with_deg(xn1, srcp, dstp)
    xs2, xn2 = _mid(xs1, aggp1, degp, W_self2, W_neigh2, b2r)
    aggp2 = _agg_only(xn2, srcp, dstp)
    return _final(xs2, aggp2, degp)
